# Initial kernel scaffold; baseline (speedup 1.0000x reference)
import jax
import jax.numpy as jnp
from jax.experimental import pallas as pl


def kernel(x, alpha, V):
    raise NotImplementedError("write your pallas kernel here")


# speedup vs baseline: 6.5524x; 6.5524x over previous
import functools

import jax
import jax.numpy as jnp
from jax.experimental import pallas as pl
from jax.experimental.pallas import tpu as pltpu

N = 2048
K = 8.0
ALPHA_LR = 0.01
NUM_ITER = 50
BATCH = 8192
BLK = 256
MAX_OUTSTANDING = 8


def _dykstra_body(alpha_ref, coef_ref):
    y0 = alpha_ref[:] * (1.0 / ALPHA_LR)

    def body(_, c):
        return c + (K - jnp.sum(jnp.clip(y0 + c, 0.0, 1.0))) / N

    c0 = (K - jnp.sum(y0)) / N
    c = jax.lax.fori_loop(0, NUM_ITER - 1, body, c0)
    coef_ref[:] = jnp.clip(y0 + c, 0.0, 1.0)


def _apply_body(coef_ref, x_ref, v_ref, y_ref, nz_ref, cnt_ref, urows_ref, sem):
    pid = pl.program_id(0)

    @pl.when(pid == 0)
    def _prep():
        cnt_ref[0] = 0

        def scan_body(d, _):
            @pl.when(coef_ref[d] != 0.0)
            def _():
                j = cnt_ref[0]
                nz_ref[j] = d
                pltpu.make_async_copy(v_ref.at[d], urows_ref.at[j], sem).start()
                cnt_ref[0] = j + 1

                @pl.when(j >= MAX_OUTSTANDING)
                def _():
                    pltpu.make_async_copy(
                        v_ref.at[0], urows_ref.at[0], sem
                    ).wait()

            return 0

        jax.lax.fori_loop(0, N, scan_body, 0)
        remaining = jnp.minimum(cnt_ref[0], MAX_OUTSTANDING)

        def drain(_, __):
            pltpu.make_async_copy(v_ref.at[0], urows_ref.at[0], sem).wait()
            return 0

        jax.lax.fori_loop(0, remaining, drain, 0)

    m = cnt_ref[0]
    y_ref[...] = jnp.zeros_like(y_ref)

    def jbody(j, _):
        d = nz_ref[j]
        row = urows_ref[pl.ds(j, 1), :]
        w = coef_ref[d] * pltpu.roll(row, d, axis=1)
        t = pltpu.roll(x_ref[...], d, axis=1)
        y_ref[...] += t * w
        return 0

    jax.lax.fori_loop(0, m, jbody, 0)


@jax.jit
def kernel(x, alpha, V):
    coef = pl.pallas_call(
        _dykstra_body,
        out_shape=jax.ShapeDtypeStruct((16, 128), jnp.float32),
    )(alpha.reshape(16, 128))
    coef = coef.reshape(N)

    y = pl.pallas_call(
        _apply_body,
        grid=(BATCH // BLK,),
        in_specs=[
            pl.BlockSpec(memory_space=pltpu.SMEM),
            pl.BlockSpec((BLK, N), lambda i: (i, 0)),
            pl.BlockSpec(memory_space=pl.ANY),
        ],
        out_specs=pl.BlockSpec((BLK, N), lambda i: (i, 0)),
        out_shape=jax.ShapeDtypeStruct((BATCH, N), jnp.float32),
        scratch_shapes=[
            pltpu.SMEM((N,), jnp.int32),
            pltpu.SMEM((1,), jnp.int32),
            pltpu.VMEM((N, N), jnp.float32),
            pltpu.SemaphoreType.DMA,
        ],
    )(coef, x, V)
    return y


# speedup vs baseline: 118.3302x; 18.0590x over previous
import jax
import jax.numpy as jnp
from jax.experimental import pallas as pl
from jax.experimental.pallas import tpu as pltpu

N = 2048
K = 8.0
ALPHA_LR = 0.01
NUM_ITER = 50
BATCH = 8192
BLK = 512
ROWCHUNK = 256


def _fused_body(alpha_ref, vt_ref, x_ref, y_ref, wt_ref):
    @pl.when(pl.program_id(0) == 0)
    def _build():
        y0 = alpha_ref[...] * (1.0 / ALPHA_LR)

        def body(_, c):
            return c + (K - jnp.sum(jnp.clip(y0 + c, 0.0, 1.0))) / N

        c = jax.lax.fori_loop(0, NUM_ITER - 1, body, (K - jnp.sum(y0)) / N)
        coef = jnp.clip(y0 + c, 0.0, 1.0)

        for k in range(N // ROWCHUNK):
            g = vt_ref[pl.ds(k * ROWCHUNK, ROWCHUNK), :] * coef
            wt = pltpu.roll(g, k * ROWCHUNK, axis=1, stride=1, stride_axis=0)
            wt_ref[pl.ds(k * ROWCHUNK, ROWCHUNK), :] = wt.astype(jnp.bfloat16)

    xb = x_ref[...].astype(jnp.bfloat16)
    y_ref[...] = jax.lax.dot_general(
        xb,
        wt_ref[...],
        (((1,), (0,)), ((), ())),
        preferred_element_type=jnp.float32,
    )


@jax.jit
def kernel(x, alpha, V):
    y = pl.pallas_call(
        _fused_body,
        grid=(BATCH // BLK,),
        in_specs=[
            pl.BlockSpec((1, N), lambda i: (0, 0)),
            pl.BlockSpec((N, N), lambda i: (0, 0)),
            pl.BlockSpec((BLK, N), lambda i: (i, 0)),
        ],
        out_specs=pl.BlockSpec((BLK, N), lambda i: (i, 0)),
        out_shape=jax.ShapeDtypeStruct((BATCH, N), jnp.float32),
        scratch_shapes=[
            pltpu.VMEM((N, N), jnp.bfloat16),
        ],
    )(alpha.reshape(1, N), V.T, x)
    return y


# speedup vs baseline: 153.9476x; 1.3010x over previous
import jax
import jax.numpy as jnp
from jax.experimental import pallas as pl
from jax.experimental.pallas import tpu as pltpu

N = 2048
K = 8.0
ALPHA_LR = 0.01
NUM_ITER = 50
BATCH = 8192
BLK = 512
COLCHUNK = 256


def _fused_body(alpha_ref, v_ref, x_ref, y_ref, w_ref):
    @pl.when(pl.program_id(0) == 0)
    def _build():
        y0 = alpha_ref[...] * (1.0 / ALPHA_LR)

        def body(_, c):
            return c + (K - jnp.sum(jnp.clip(y0 + c, 0.0, 1.0))) / N

        c = jax.lax.fori_loop(0, NUM_ITER - 1, body, (K - jnp.sum(y0)) / N)
        coef = jnp.clip(y0 + c, 0.0, 1.0)

        for k in range(N // COLCHUNK):
            gt = jnp.transpose(v_ref[:, pl.ds(k * COLCHUNK, COLCHUNK)]) * coef
            w = pltpu.roll(gt, k * COLCHUNK, axis=1, stride=1, stride_axis=0)
            w_ref[pl.ds(k * COLCHUNK, COLCHUNK), :] = w.astype(jnp.bfloat16)

    xb = x_ref[...].astype(jnp.bfloat16)
    y_ref[...] = jax.lax.dot_general(
        xb,
        w_ref[...],
        (((1,), (0,)), ((), ())),
        preferred_element_type=jnp.float32,
    )


@jax.jit
def kernel(x, alpha, V):
    y = pl.pallas_call(
        _fused_body,
        grid=(BATCH // BLK,),
        in_specs=[
            pl.BlockSpec((1, N), lambda i: (0, 0)),
            pl.BlockSpec((N, N), lambda i: (0, 0)),
            pl.BlockSpec((BLK, N), lambda i: (i, 0)),
        ],
        out_specs=pl.BlockSpec((BLK, N), lambda i: (i, 0)),
        out_shape=jax.ShapeDtypeStruct((BATCH, N), jnp.float32),
        scratch_shapes=[
            pltpu.VMEM((N, N), jnp.bfloat16),
        ],
    )(alpha.reshape(1, N), V, x)
    return y


# speedup vs baseline: 154.1537x; 1.0013x over previous
import jax
import jax.numpy as jnp
from jax.experimental import pallas as pl
from jax.experimental.pallas import tpu as pltpu

N = 2048
K = 8.0
ALPHA_LR = 0.01
NUM_ITER = 50
BATCH = 8192
BLK = 1024
CHUNK = 256
NCHUNK = N // CHUNK


def _fused_body(alpha_ref, v_any, x_ref, y_ref, wt_ref, stage_ref, sems):
    @pl.when(pl.program_id(0) == 0)
    def _build():
        y0 = alpha_ref[...] * (1.0 / ALPHA_LR)

        def body(_, c):
            return c + (K - jnp.sum(jnp.clip(y0 + c, 0.0, 1.0))) / N

        c = jax.lax.fori_loop(0, NUM_ITER - 1, body, (K - jnp.sum(y0)) / N)
        coef = jnp.clip(y0 + c, 0.0, 1.0)

        def copy_in(k):
            return pltpu.make_async_copy(
                v_any.at[:, pl.ds(k * CHUNK, CHUNK)],
                stage_ref.at[k % 2],
                sems.at[k % 2],
            )

        copy_in(0).start()
        for k in range(NCHUNK):
            if k + 1 < NCHUNK:
                copy_in(k + 1).start()
            copy_in(k).wait()
            gt = jnp.transpose(stage_ref[k % 2]) * coef
            w = pltpu.roll(gt, k * CHUNK, axis=1, stride=1, stride_axis=0)
            wt_ref[pl.ds(k * CHUNK, CHUNK), :] = w.astype(jnp.bfloat16)

    xb = x_ref[...].astype(jnp.bfloat16)
    y_ref[...] = jax.lax.dot_general(
        xb,
        wt_ref[...],
        (((1,), (0,)), ((), ())),
        preferred_element_type=jnp.float32,
    )


@jax.jit
def kernel(x, alpha, V):
    y = pl.pallas_call(
        _fused_body,
        grid=(BATCH // BLK,),
        in_specs=[
            pl.BlockSpec((1, N), lambda i: (0, 0)),
            pl.BlockSpec(memory_space=pl.ANY),
            pl.BlockSpec((BLK, N), lambda i: (i, 0)),
        ],
        out_specs=pl.BlockSpec((BLK, N), lambda i: (i, 0)),
        out_shape=jax.ShapeDtypeStruct((BATCH, N), jnp.float32),
        scratch_shapes=[
            pltpu.VMEM((N, N), jnp.bfloat16),
            pltpu.VMEM((2, N, CHUNK), jnp.float32),
            pltpu.SemaphoreType.DMA((2,)),
        ],
    )(alpha.reshape(1, N), V, x)
    return y
